# baseline (speedup 1.0000x reference)
import jax
import jax.numpy as jnp
from jax.experimental import pallas as pl
from jax.experimental.pallas import tpu as pltpu


def _cell_body(tmap_ref, fin_ref, ufc_ref, uiouc_ref, bfr_ref, biour_ref,
               nh_hbm, nc_hbm, iou_hbm, c_hbm,
               nh_buf, nc_buf, iou_st, c_st, in_sems, out_sems):
    N, K, H = nh_hbm.shape
    NT = bfr_ref.shape[1] // H
    CH = nh_buf.shape[1]
    NCH = N // CH
    O = 3 * H
    neg_log2e = -1.4426950408889634

    def in_copies(i, slot):
        return (pltpu.make_async_copy(nh_hbm.at[pl.ds(i * CH, CH)],
                                      nh_buf.at[slot], in_sems.at[slot, 0]),
                pltpu.make_async_copy(nc_hbm.at[pl.ds(i * CH, CH)],
                                      nc_buf.at[slot], in_sems.at[slot, 1]))

    def out_copies(i, slot):
        return (pltpu.make_async_copy(iou_st.at[slot],
                                      iou_hbm.at[pl.ds(i * CH, CH)],
                                      out_sems.at[slot, 0]),
                pltpu.make_async_copy(c_st.at[slot],
                                      c_hbm.at[pl.ds(i * CH, CH)],
                                      out_sems.at[slot, 1]))

    for c in in_copies(0, 0):
        c.start()

    def step(i, carry):
        slot = jax.lax.rem(i, 2)

        @pl.when(i + 1 < NCH)
        def _prefetch():
            for c in in_copies(i + 1, 1 - slot):
                c.start()

        for c in in_copies(i, slot):
            c.wait()

        @pl.when(i >= 2)
        def _drain_prev():
            for c in out_copies(i - 2, slot):
                c.wait()

        nh = nh_buf[slot]
        nc = nc_buf[slot]
        fin = fin_ref[pl.ds(i * CH, CH)]
        tmap = tmap_ref[pl.ds(i * CH, CH)]

        h_tilde = jnp.sum(nh, axis=1)

        piou = jnp.dot(h_tilde.astype(jnp.bfloat16), uiouc_ref[...],
                       preferred_element_type=jnp.float32)
        t1 = tmap[:, :1]
        iou_sel = piou[:, 0:O]
        biou_sel = biour_ref[0:1, 0:O]
        for t in range(1, NT):
            cond = t1 == t
            iou_sel = jnp.where(cond, piou[:, t * O:(t + 1) * O], iou_sel)
            biou_sel = jnp.where(cond, biour_ref[0:1, t * O:(t + 1) * O],
                                 biou_sel)
        iou_st[slot] = iou_sel + biou_sel

        pf = jnp.dot(nh.reshape(CH * K, H).astype(jnp.bfloat16),
                     ufc_ref[...],
                     preferred_element_type=jnp.float32)
        pf = pf.reshape(CH, K, NT * H)
        tb = jnp.broadcast_to(tmap[:, :1], (CH, H))
        cond3 = tb[:, None, :]
        f_sel = pf[:, :, 0:H]
        bf_sel = bfr_ref[0:1, 0:H]
        for t in range(1, NT):
            f_sel = jnp.where(cond3 == t, pf[:, :, t * H:(t + 1) * H], f_sel)
            bf_sel = jnp.where(t1 == t, bfr_ref[0:1, t * H:(t + 1) * H],
                               bf_sel)
        ys = f_sel + ((fin + bf_sel) * neg_log2e)[:, None, :]
        gate = 1.0 / (1.0 + jnp.exp2(ys))
        c_st[slot] = jnp.sum(gate * nc, axis=1)

        for c in out_copies(i, slot):
            c.start()
        return carry

    jax.lax.fori_loop(0, NCH, step, 0)
    for i in range(max(0, NCH - 2), NCH):
        for c in out_copies(i, i % 2):
            c.wait()


def kernel(n_h, n_c, f_in, type_id, U_iou, b_iou, U_f, b_f):
    N, K, H = n_h.shape
    NT = U_iou.shape[0]
    CH = max((b for b in range(8, min(200, N) + 1, 8) if N % b == 0),
             default=N)

    tmap = jnp.broadcast_to(type_id.astype(jnp.int32)[:, None], (N, 8))
    ufc = (U_f.transpose(1, 0, 2).reshape(H, NT * H)
           * (-1.4426950408889634)).astype(jnp.bfloat16)
    uiouc = U_iou.transpose(1, 0, 2).reshape(H, NT * 3 * H).astype(jnp.bfloat16)
    bfr = jnp.tile(b_f.reshape(1, NT * H), (8, 1))
    biour = jnp.tile(b_iou.reshape(1, NT * 3 * H), (8, 1))

    vmem = pl.BlockSpec(memory_space=pltpu.VMEM)
    hbm = pl.BlockSpec(memory_space=pl.ANY)
    iou_aggr, c_aggr = pl.pallas_call(
        _cell_body,
        in_specs=[vmem, vmem, vmem, vmem, vmem, vmem, hbm, hbm],
        out_specs=[hbm, hbm],
        out_shape=[
            jax.ShapeDtypeStruct((N, 3 * H), n_h.dtype),
            jax.ShapeDtypeStruct((N, H), n_h.dtype),
        ],
        scratch_shapes=[
            pltpu.VMEM((2, CH, K, H), jnp.float32),
            pltpu.VMEM((2, CH, K, H), jnp.float32),
            pltpu.VMEM((2, CH, 3 * H), jnp.float32),
            pltpu.VMEM((2, CH, H), jnp.float32),
            pltpu.SemaphoreType.DMA((2, 2)),
            pltpu.SemaphoreType.DMA((2, 2)),
        ],
    )(tmap, f_in, ufc, uiouc, bfr, biour, n_h, n_c)
    return iou_aggr, c_aggr


# speedup vs baseline: 1.1261x; 1.1261x over previous
import jax
import jax.numpy as jnp
from jax.experimental import pallas as pl
from jax.experimental.pallas import tpu as pltpu


def _cell_body(tmap_ref, fin_ref, ufc_ref, uiouc_ref, bfr_ref, biour_ref,
               nh_hbm, nc_hbm, iou_hbm, c_hbm,
               nh_buf, nc_buf, iou_st, c_st, in_sems, out_sems):
    N, K, H = nh_hbm.shape
    NT = bfr_ref.shape[1] // H
    CH = nh_buf.shape[1]
    NCH = N // CH
    O = 3 * H
    neg_log2e = -1.4426950408889634

    def in_copies(i, slot):
        return (pltpu.make_async_copy(nh_hbm.at[pl.ds(i * CH, CH)],
                                      nh_buf.at[slot], in_sems.at[slot, 0]),
                pltpu.make_async_copy(nc_hbm.at[pl.ds(i * CH, CH)],
                                      nc_buf.at[slot], in_sems.at[slot, 1]))

    def out_copies(i, slot):
        return (pltpu.make_async_copy(iou_st.at[slot],
                                      iou_hbm.at[pl.ds(i * CH, CH)],
                                      out_sems.at[slot, 0]),
                pltpu.make_async_copy(c_st.at[slot],
                                      c_hbm.at[pl.ds(i * CH, CH)],
                                      out_sems.at[slot, 1]))

    NSLOT = nh_buf.shape[0]
    for j in range(min(NSLOT - 1, NCH)):
        for c in in_copies(j, j):
            c.start()

    def step(i, carry):
        slot = jax.lax.rem(i, NSLOT)

        @pl.when(i + NSLOT - 1 < NCH)
        def _prefetch():
            for c in in_copies(i + NSLOT - 1,
                               jax.lax.rem(i + NSLOT - 1, NSLOT)):
                c.start()

        for c in in_copies(i, slot):
            c.wait()

        oslot = jax.lax.rem(i, 2)

        @pl.when(i >= 2)
        def _drain_prev():
            for c in out_copies(i - 2, oslot):
                c.wait()

        nh = nh_buf[slot]
        nc = nc_buf[slot]
        fin = fin_ref[pl.ds(i * CH, CH)]
        tmap = tmap_ref[pl.ds(i * CH, CH)]

        h_tilde = jnp.sum(nh, axis=1)

        piou = jnp.dot(h_tilde.astype(jnp.bfloat16), uiouc_ref[...],
                       preferred_element_type=jnp.float32)
        t1 = tmap[:, :1]
        iou_sel = piou[:, 0:O]
        biou_sel = biour_ref[0:1, 0:O]
        for t in range(1, NT):
            cond = t1 == t
            iou_sel = jnp.where(cond, piou[:, t * O:(t + 1) * O], iou_sel)
            biou_sel = jnp.where(cond, biour_ref[0:1, t * O:(t + 1) * O],
                                 biou_sel)
        iou_st[oslot] = iou_sel + biou_sel

        pf = jnp.dot(nh.reshape(CH * K, H).astype(jnp.bfloat16),
                     ufc_ref[...],
                     preferred_element_type=jnp.float32)
        pf = pf.reshape(CH, K, NT * H)
        tb = jnp.broadcast_to(tmap[:, :1], (CH, H))
        cond3 = tb[:, None, :]
        f_sel = pf[:, :, 0:H]
        bf_sel = bfr_ref[0:1, 0:H]
        for t in range(1, NT):
            f_sel = jnp.where(cond3 == t, pf[:, :, t * H:(t + 1) * H], f_sel)
            bf_sel = jnp.where(t1 == t, bfr_ref[0:1, t * H:(t + 1) * H],
                               bf_sel)
        ys = f_sel + ((fin + bf_sel) * neg_log2e)[:, None, :]
        gate = 1.0 / (1.0 + jnp.exp2(ys))
        c_st[oslot] = jnp.sum(gate * nc, axis=1)

        for c in out_copies(i, oslot):
            c.start()
        return carry

    jax.lax.fori_loop(0, NCH, step, 0)
    for i in range(max(0, NCH - 2), NCH):
        for c in out_copies(i, i % 2):
            c.wait()


def kernel(n_h, n_c, f_in, type_id, U_iou, b_iou, U_f, b_f):
    N, K, H = n_h.shape
    NT = U_iou.shape[0]
    CH = max((b for b in range(8, min(200, N) + 1, 8) if N % b == 0),
             default=N)

    tmap = jnp.broadcast_to(type_id.astype(jnp.int32)[:, None], (N, 8))
    ufc = (U_f.transpose(1, 0, 2).reshape(H, NT * H)
           * (-1.4426950408889634)).astype(jnp.bfloat16)
    uiouc = U_iou.transpose(1, 0, 2).reshape(H, NT * 3 * H).astype(jnp.bfloat16)
    bfr = jnp.tile(b_f.reshape(1, NT * H), (8, 1))
    biour = jnp.tile(b_iou.reshape(1, NT * 3 * H), (8, 1))

    vmem = pl.BlockSpec(memory_space=pltpu.VMEM)
    hbm = pl.BlockSpec(memory_space=pl.ANY)
    iou_aggr, c_aggr = pl.pallas_call(
        _cell_body,
        in_specs=[vmem, vmem, vmem, vmem, vmem, vmem, hbm, hbm],
        out_specs=[hbm, hbm],
        out_shape=[
            jax.ShapeDtypeStruct((N, 3 * H), n_h.dtype),
            jax.ShapeDtypeStruct((N, H), n_h.dtype),
        ],
        scratch_shapes=[
            pltpu.VMEM((3, CH, K, H), jnp.float32),
            pltpu.VMEM((3, CH, K, H), jnp.float32),
            pltpu.VMEM((2, CH, 3 * H), jnp.float32),
            pltpu.VMEM((2, CH, H), jnp.float32),
            pltpu.SemaphoreType.DMA((3, 2)),
            pltpu.SemaphoreType.DMA((2, 2)),
        ],
    )(tmap, f_in, ufc, uiouc, bfr, biour, n_h, n_c)
    return iou_aggr, c_aggr


# speedup vs baseline: 1.1373x; 1.0099x over previous
import jax
import jax.numpy as jnp
from jax.experimental import pallas as pl
from jax.experimental.pallas import tpu as pltpu


def _cell_body(tmap_ref, fin_ref, ufc_ref, uiouc_ref, bfr_ref, biour_ref,
               nh_hbm, nc_hbm, iou_hbm, c_hbm,
               nh_buf, nc_buf, iou_st, c_st, in_sems, out_sems):
    N, K, H = nh_hbm.shape
    NT = bfr_ref.shape[1] // H
    CH = nh_buf.shape[1]
    NCH = N // CH
    O = 3 * H
    neg_log2e = -1.4426950408889634

    def in_copies(i, slot):
        return (pltpu.make_async_copy(nh_hbm.at[pl.ds(i * CH, CH)],
                                      nh_buf.at[slot], in_sems.at[slot, 0]),
                pltpu.make_async_copy(nc_hbm.at[pl.ds(i * CH, CH)],
                                      nc_buf.at[slot], in_sems.at[slot, 1]))

    def out_copies(i, slot):
        return (pltpu.make_async_copy(iou_st.at[slot],
                                      iou_hbm.at[pl.ds(i * CH, CH)],
                                      out_sems.at[slot, 0]),
                pltpu.make_async_copy(c_st.at[slot],
                                      c_hbm.at[pl.ds(i * CH, CH)],
                                      out_sems.at[slot, 1]))

    NSLOT = nh_buf.shape[0]
    for j in range(min(NSLOT - 1, NCH)):
        for c in in_copies(j, j):
            c.start()

    def step(i, carry):
        slot = jax.lax.rem(i, NSLOT)

        @pl.when(i + NSLOT - 1 < NCH)
        def _prefetch():
            for c in in_copies(i + NSLOT - 1,
                               jax.lax.rem(i + NSLOT - 1, NSLOT)):
                c.start()

        for c in in_copies(i, slot):
            c.wait()

        oslot = jax.lax.rem(i, 2)

        @pl.when(i >= 2)
        def _drain_prev():
            for c in out_copies(i - 2, oslot):
                c.wait()

        nh = nh_buf[slot]
        nc = nc_buf[slot]
        fin = fin_ref[pl.ds(i * CH, CH)]
        tmap = tmap_ref[pl.ds(i * CH, CH)]

        h_tilde = jnp.sum(nh, axis=1)

        piou = jnp.dot(h_tilde.astype(jnp.bfloat16), uiouc_ref[...],
                       preferred_element_type=jnp.float32)
        t1 = tmap[:, :1]
        iou_sel = piou[:, 0:O]
        biou_sel = biour_ref[0:1, 0:O]
        for t in range(1, NT):
            cond = t1 == t
            iou_sel = jnp.where(cond, piou[:, t * O:(t + 1) * O], iou_sel)
            biou_sel = jnp.where(cond, biour_ref[0:1, t * O:(t + 1) * O],
                                 biou_sel)
        iou_st[oslot] = iou_sel + biou_sel

        pf = jnp.dot(nh.reshape(CH * K, H).astype(jnp.bfloat16),
                     ufc_ref[...],
                     preferred_element_type=jnp.float32)
        pf = pf.reshape(CH, K, NT * H)
        tb = jnp.broadcast_to(tmap[:, :1], (CH, H))
        cond3 = tb[:, None, :]
        f_sel = pf[:, :, 0:H]
        bf_sel = bfr_ref[0:1, 0:H]
        for t in range(1, NT):
            f_sel = jnp.where(cond3 == t, pf[:, :, t * H:(t + 1) * H], f_sel)
            bf_sel = jnp.where(t1 == t, bfr_ref[0:1, t * H:(t + 1) * H],
                               bf_sel)
        ys = f_sel + ((fin + bf_sel) * neg_log2e)[:, None, :]
        gate = 1.0 / (1.0 + jnp.exp2(ys))
        c_st[oslot] = jnp.sum(gate * nc, axis=1)

        for c in out_copies(i, oslot):
            c.start()
        return carry

    jax.lax.fori_loop(0, NCH, step, 0)
    for i in range(max(0, NCH - 2), NCH):
        for c in out_copies(i, i % 2):
            c.wait()


def kernel(n_h, n_c, f_in, type_id, U_iou, b_iou, U_f, b_f):
    N, K, H = n_h.shape
    NT = U_iou.shape[0]
    CH = max((b for b in range(8, min(200, N) + 1, 8) if N % b == 0),
             default=N)

    tmap = jnp.broadcast_to(type_id.astype(jnp.int32)[:, None], (N, 8))
    ufc = (U_f.transpose(1, 0, 2).reshape(H, NT * H)
           * (-1.4426950408889634)).astype(jnp.bfloat16)
    uiouc = U_iou.transpose(1, 0, 2).reshape(H, NT * 3 * H).astype(jnp.bfloat16)
    bfr = jnp.tile(b_f.reshape(1, NT * H), (8, 1))
    biour = jnp.tile(b_iou.reshape(1, NT * 3 * H), (8, 1))

    vmem = pl.BlockSpec(memory_space=pltpu.VMEM)
    hbm = pl.BlockSpec(memory_space=pl.ANY)
    iou_aggr, c_aggr = pl.pallas_call(
        _cell_body,
        in_specs=[vmem, vmem, vmem, vmem, vmem, vmem, hbm, hbm],
        out_specs=[hbm, hbm],
        out_shape=[
            jax.ShapeDtypeStruct((N, 3 * H), n_h.dtype),
            jax.ShapeDtypeStruct((N, H), n_h.dtype),
        ],
        scratch_shapes=[
            pltpu.VMEM((4, CH, K, H), jnp.float32),
            pltpu.VMEM((4, CH, K, H), jnp.float32),
            pltpu.VMEM((2, CH, 3 * H), jnp.float32),
            pltpu.VMEM((2, CH, H), jnp.float32),
            pltpu.SemaphoreType.DMA((4, 2)),
            pltpu.SemaphoreType.DMA((2, 2)),
        ],
    )(tmap, f_in, ufc, uiouc, bfr, biour, n_h, n_c)
    return iou_aggr, c_aggr
